# Initial kernel scaffold; baseline (speedup 1.0000x reference)
import jax
import jax.numpy as jnp
from jax.experimental import pallas as pl


def kernel(x, tables):
    raise NotImplementedError("write your pallas kernel here")


# speedup vs baseline: 1.7180x; 1.7180x over previous
import functools

import jax
import jax.numpy as jnp
from jax import lax
from jax.experimental import pallas as pl
from jax.experimental.pallas import tpu as pltpu
from jax.experimental.pallas import tpu_sc as plsc

_INPUT_DIM = 100
_N_CONT = 74
_N_CAT = 26
_CAT_DIM = 100000
_EMB_DIM = 16
_BATCH = 16384
_OUT_DIM = _N_CONT + _N_CAT * _EMB_DIM

_NC, _NS = 2, 16
_NW = _NC * _NS
_ROWS_PER_W = _BATCH // _NW
_CHUNK = 128
_N_CHUNKS = _ROWS_PER_W // _CHUNK


def _body(idx_hbm, xf_hbm, tables_hbm, out_hbm, idx_v, emb_v, cont_v, out_v,
          sem, csem):
    wid = lax.axis_index("s") * _NC + lax.axis_index("c")
    base = wid * _ROWS_PER_W
    for step in range(_N_CHUNKS):
        cbase = base + step * _CHUNK
        pltpu.sync_copy(idx_hbm.at[pl.ds(cbase * _N_CAT, _CHUNK * _N_CAT)],
                        idx_v)
        gat = pltpu.async_copy(tables_hbm.at[idx_v], emb_v, sem)
        cont = pltpu.async_copy(xf_hbm.at[pl.ds(cbase, _CHUNK)], cont_v, csem)
        cont.wait()
        gat.wait()

        def place_row(r, _):
            out_v[r, pl.ds(_N_CONT - 16, 16)] = cont_v[r, pl.ds(_N_CONT - 16, 16)]
            for k in range(4):
                out_v[r, pl.ds(16 * k, 16)] = cont_v[r, pl.ds(16 * k, 16)]
            for c in range(_N_CAT):
                out_v[r, pl.ds(_N_CONT + 16 * c, 16)] = emb_v[r * _N_CAT + c, :]
            return 0

        lax.fori_loop(0, _CHUNK, place_row, 0)
        pltpu.sync_copy(out_v, out_hbm.at[pl.ds(cbase, _CHUNK)])


@jax.jit
def kernel(x, tables):
    xf = jnp.pad(x[:, :_N_CONT].astype(jnp.float32), ((0, 0), (0, 6)))
    flat_idx = (x[:, _N_CONT:]
                + jnp.arange(_N_CAT, dtype=jnp.int32) * _CAT_DIM).reshape(-1)
    tables2d = tables.reshape(_N_CAT * _CAT_DIM, _EMB_DIM)
    run = functools.partial(
        pl.kernel,
        out_type=jax.ShapeDtypeStruct((_BATCH, _OUT_DIM), jnp.float32),
        mesh=plsc.VectorSubcoreMesh(core_axis_name="c", subcore_axis_name="s"),
        scratch_types=[
            pltpu.VMEM((_CHUNK * _N_CAT,), jnp.int32),
            pltpu.VMEM((_CHUNK * _N_CAT, _EMB_DIM), jnp.float32),
            pltpu.VMEM((_CHUNK, _N_CONT + 6), jnp.float32),
            pltpu.VMEM((_CHUNK, _OUT_DIM), jnp.float32),
            pltpu.SemaphoreType.DMA,
            pltpu.SemaphoreType.DMA,
        ],
        compiler_params=pltpu.CompilerParams(use_tc_tiling_on_sc=False),
    )(_body)
    return run(flat_idx, xf, tables2d)


# speedup vs baseline: 1.7347x; 1.0097x over previous
import functools

import jax
import jax.numpy as jnp
from jax import lax
from jax.experimental import pallas as pl
from jax.experimental.pallas import tpu as pltpu
from jax.experimental.pallas import tpu_sc as plsc

_INPUT_DIM = 100
_N_CONT = 74
_N_CAT = 26
_CAT_DIM = 100000
_EMB_DIM = 16
_BATCH = 16384
_OUT_DIM = _N_CONT + _N_CAT * _EMB_DIM

_NC, _NS = 2, 16
_NW = _NC * _NS
_ROWS_PER_W = _BATCH // _NW
_CHUNK = 128
_N_CHUNKS = _ROWS_PER_W // _CHUNK


def _body(idx_hbm, xf_hbm, tables_hbm, out_hbm, idx_v, emb_v, tail_v, out_v,
          sem, csem):
    wid = lax.axis_index("s") * _NC + lax.axis_index("c")
    base = wid * _ROWS_PER_W
    for step in range(_N_CHUNKS):
        cbase = base + step * _CHUNK
        pltpu.sync_copy(idx_hbm.at[:, pl.ds(cbase, _CHUNK)], idx_v)
        gathers = [
            pltpu.async_copy(tables_hbm.at[c].at[idx_v.at[c]], emb_v.at[c],
                             sem)
            for c in range(_N_CAT)
        ]
        cont = pltpu.async_copy(
            xf_hbm.at[pl.ds(cbase, _CHUNK), pl.ds(0, 72)],
            out_v.at[:, pl.ds(0, 72)], csem)
        cont2 = pltpu.async_copy(
            xf_hbm.at[pl.ds(cbase, _CHUNK), pl.ds(64, 16)], tail_v, csem)
        cont.wait()
        cont2.wait()
        for g in gathers:
            g.wait()

        def place_row(r, _):
            out_v[r, pl.ds(64, 16)] = tail_v[r, :]
            for c in range(_N_CAT):
                out_v[r, pl.ds(_N_CONT + 16 * c, 16)] = emb_v[c, r, :]
            return 0

        lax.fori_loop(0, _CHUNK, place_row, 0)
        pltpu.sync_copy(out_v, out_hbm.at[pl.ds(cbase, _CHUNK)])


@jax.jit
def kernel(x, tables):
    xf = jnp.pad(x[:, :_N_CONT].astype(jnp.float32), ((0, 0), (0, 6)))
    idx_t = x[:, _N_CONT:].T
    run = functools.partial(
        pl.kernel,
        out_type=jax.ShapeDtypeStruct((_BATCH, _OUT_DIM), jnp.float32),
        mesh=plsc.VectorSubcoreMesh(core_axis_name="c", subcore_axis_name="s"),
        scratch_types=[
            pltpu.VMEM((_N_CAT, _CHUNK), jnp.int32),
            pltpu.VMEM((_N_CAT, _CHUNK, _EMB_DIM), jnp.float32),
            pltpu.VMEM((_CHUNK, _EMB_DIM), jnp.float32),
            pltpu.VMEM((_CHUNK, _OUT_DIM), jnp.float32),
            pltpu.SemaphoreType.DMA,
            pltpu.SemaphoreType.DMA,
        ],
        compiler_params=pltpu.CompilerParams(use_tc_tiling_on_sc=False),
    )(_body)
    return run(idx_t, xf, tables)
